# baseline (speedup 1.0000x reference)
import math

import jax
import jax.numpy as jnp
from jax import lax
from jax.experimental import pallas as pl
from jax.experimental.pallas import tpu as pltpu
from jax.experimental.pallas import tpu_sc as plsc

VOCAB = 100000
EMB = 128
B = 1024
S = 200
SCALE = math.sqrt(EMB)

NC, NS = 2, 16
NW = NC * NS
ROWS = B * S
ROWS_PER_W = ROWS // NW
CHUNK = 128
NCHUNK = ROWS_PER_W // CHUNK
NBUF = 5
LANES = 16
SUB = EMB // LANES


def _sc_body(table_hbm, idx_hbm, pe_hbm, out_hbm, idx_v, pe_v, bufs, *sems):
    sem_in, sem_out = sems[:NBUF], sems[NBUF:]
    wid = lax.axis_index("s") * NC + lax.axis_index("c")
    base = wid * ROWS_PER_W

    pltpu.sync_copy(idx_hbm.at[wid], idx_v)
    pltpu.sync_copy(pe_hbm, pe_v)

    def gather_start(c, b):
        pltpu.async_copy(table_hbm.at[idx_v.at[c]], bufs.at[b], sem_in[b])

    def gather_wait(b):
        pltpu.make_async_copy(
            out_hbm.at[pl.ds(0, CHUNK)], bufs.at[b], sem_in[b]
        ).wait()

    def out_start(c, b):
        pltpu.async_copy(
            bufs.at[b], out_hbm.at[pl.ds(base + c * CHUNK, CHUNK)], sem_out[b]
        )

    def out_wait(b):
        pltpu.make_async_copy(
            bufs.at[b], out_hbm.at[pl.ds(0, CHUNK)], sem_out[b]
        ).wait()

    def compute(c, b):
        buf = bufs.at[b]

        @plsc.parallel_loop(0, CHUNK, 1, unroll=4)
        def _(i):
            s = lax.rem(c * CHUNK + i, S)
            for h in range(SUB):
                sl = pl.ds(h * LANES, LANES)
                buf[i, sl] = buf[i, sl] * SCALE + pe_v[s, sl]

    for c in range(NBUF - 1):
        gather_start(c, c)

    @pl.loop(0, NCHUNK // NBUF)
    def _(g):
        c0 = g * NBUF
        for j in range(NBUF):
            c = c0 + j
            gather_wait(j)
            compute(c, j)
            if j % 2 == 1:
                out_start(c - 1, j - 1)
                out_start(c, j)
            elif j == NBUF - 1:
                out_start(c, j)
            d = c + NBUF - 1
            bd = (j + NBUF - 1) % NBUF
            if j == 0:
                @pl.when(g > 0)
                def _():
                    out_wait(bd)
                    gather_start(d, bd)

                @pl.when(g == 0)
                def _():
                    gather_start(d, bd)
            else:
                @pl.when(d < NCHUNK)
                def _():
                    out_wait(bd)
                    gather_start(d, bd)

    for b in range(NBUF):
        out_wait(b)


@jax.jit
def _embed(table, idx, pe):
    mesh = plsc.VectorSubcoreMesh(
        core_axis_name="c", subcore_axis_name="s", num_cores=NC, num_subcores=NS
    )
    return pl.kernel(
        _sc_body,
        out_type=jax.ShapeDtypeStruct((ROWS, EMB), jnp.float32),
        mesh=mesh,
        scratch_types=[
            pltpu.VMEM((NCHUNK, CHUNK), jnp.int32),
            pltpu.VMEM((S, EMB), jnp.float32),
            pltpu.VMEM((NBUF, CHUNK, EMB), jnp.float32),
        ] + [pltpu.SemaphoreType.DMA] * (2 * NBUF),
    )(table, idx, pe)


def kernel(token_sequences, embedding_weight, positional_embedding):
    idx = token_sequences.astype(jnp.int32).reshape(NW, NCHUNK, CHUNK)
    pe = positional_embedding[0, :S, :]
    out = _embed(embedding_weight, idx, pe)
    return out.reshape(B, S, EMB)


# speedup vs baseline: 1.0936x; 1.0936x over previous
import math

import jax
import jax.numpy as jnp
from jax import lax
from jax.experimental import pallas as pl
from jax.experimental.pallas import tpu as pltpu
from jax.experimental.pallas import tpu_sc as plsc

VOCAB = 100000
EMB = 128
B = 1024
S = 200
SCALE = math.sqrt(EMB)

NC, NS = 2, 16
NW = NC * NS
ROWS = B * S
ROWS_PER_W = ROWS // NW
CHUNK = 128
NCHUNK = ROWS_PER_W // CHUNK
NBUF = 5
LANES = 16
SUB = EMB // LANES


def _sc_body(table_hbm, idx_hbm, pe_hbm, out_hbm, idx_v, pe_v, bufs, *sems):
    sem_in, sem_out = sems[:NBUF], sems[NBUF:]
    wid = lax.axis_index("s") * NC + lax.axis_index("c")
    base = wid * ROWS_PER_W

    pltpu.sync_copy(idx_hbm.at[wid], idx_v)
    pltpu.sync_copy(pe_hbm, pe_v)

    def gather_start(c, b):
        pltpu.async_copy(table_hbm.at[idx_v.at[c]], bufs.at[b], sem_in[b])

    def gather_wait(b):
        pltpu.make_async_copy(
            out_hbm.at[pl.ds(0, CHUNK)], bufs.at[b], sem_in[b]
        ).wait()

    def out_start(c, b):
        pltpu.async_copy(
            bufs.at[b], out_hbm.at[pl.ds(base + c * CHUNK, CHUNK)], sem_out[b]
        )

    def out_wait(b):
        pltpu.make_async_copy(
            bufs.at[b], out_hbm.at[pl.ds(0, CHUNK)], sem_out[b]
        ).wait()

    def compute(c, b):
        buf = bufs.at[b]

        @plsc.parallel_loop(0, CHUNK, 1, unroll=4)
        def _(i):
            s = lax.rem(c * CHUNK + i, S)
            for h in range(SUB):
                sl = pl.ds(h * LANES, LANES)
                buf[i, sl] = buf[i, sl] * SCALE + pe_v[s, sl]

    for c in range(NBUF - 1):
        gather_start(c, c)

    @pl.loop(0, NCHUNK // NBUF)
    def _(g):
        c0 = g * NBUF
        for j in range(NBUF):
            c = c0 + j
            gather_wait(j)
            compute(c, j)
            out_start(c, j)
            d = c + NBUF - 1
            bd = (j + NBUF - 1) % NBUF
            if j == 0:
                @pl.when(g > 0)
                def _():
                    out_wait(bd)
                    gather_start(d, bd)

                @pl.when(g == 0)
                def _():
                    gather_start(d, bd)
            else:
                @pl.when(d < NCHUNK)
                def _():
                    out_wait(bd)
                    gather_start(d, bd)

    for b in range(NBUF):
        out_wait(b)


@jax.jit
def _embed(table, idx, pe):
    mesh = plsc.VectorSubcoreMesh(
        core_axis_name="c", subcore_axis_name="s", num_cores=NC, num_subcores=NS
    )
    return pl.kernel(
        _sc_body,
        out_type=jax.ShapeDtypeStruct((ROWS, EMB), jnp.float32),
        mesh=mesh,
        scratch_types=[
            pltpu.VMEM((NCHUNK, CHUNK), jnp.int32),
            pltpu.VMEM((S, EMB), jnp.float32),
            pltpu.VMEM((NBUF, CHUNK, EMB), jnp.float32),
        ] + [pltpu.SemaphoreType.DMA] * (2 * NBUF),
    )(table, idx, pe)


def kernel(token_sequences, embedding_weight, positional_embedding):
    idx = token_sequences.astype(jnp.int32).reshape(NW, NCHUNK, CHUNK)
    pe = positional_embedding[0, :S, :]
    out = _embed(embedding_weight, idx, pe)
    return out.reshape(B, S, EMB)


# speedup vs baseline: 1.1055x; 1.0109x over previous
import math

import jax
import jax.numpy as jnp
from jax import lax
from jax.experimental import pallas as pl
from jax.experimental.pallas import tpu as pltpu
from jax.experimental.pallas import tpu_sc as plsc

VOCAB = 100000
EMB = 128
B = 1024
S = 200
SCALE = math.sqrt(EMB)

NC, NS = 2, 16
NW = NC * NS
ROWS = B * S
ROWS_PER_W = ROWS // NW
CHUNK = 128
NCHUNK = ROWS_PER_W // CHUNK
NBUF = 5
LANES = 16
SUB = EMB // LANES


def _sc_body(table_hbm, idx_hbm, pe_hbm, out_hbm, idx_v, pe_v, bufs, *sems):
    sem_in, sem_out, sem_pe = sems[:NBUF], sems[NBUF:2 * NBUF], sems[2 * NBUF]
    wid = lax.axis_index("s") * NC + lax.axis_index("c")
    base = wid * ROWS_PER_W

    pltpu.sync_copy(idx_hbm.at[wid], idx_v)
    pe_copy = pltpu.async_copy(pe_hbm, pe_v, sem_pe)

    def gather_start(c, b):
        pltpu.async_copy(table_hbm.at[idx_v.at[c]], bufs.at[b], sem_in[b])

    def gather_wait(b):
        pltpu.make_async_copy(
            out_hbm.at[pl.ds(0, CHUNK)], bufs.at[b], sem_in[b]
        ).wait()

    def out_start(c, b):
        pltpu.async_copy(
            bufs.at[b], out_hbm.at[pl.ds(base + c * CHUNK, CHUNK)], sem_out[b]
        )

    def out_wait(b):
        pltpu.make_async_copy(
            bufs.at[b], out_hbm.at[pl.ds(0, CHUNK)], sem_out[b]
        ).wait()

    def compute(c, b):
        buf = bufs.at[b]

        @plsc.parallel_loop(0, CHUNK, 1, unroll=4)
        def _(i):
            s = lax.rem(c * CHUNK + i, S)
            for h in range(SUB):
                sl = pl.ds(h * LANES, LANES)
                buf[i, sl] = buf[i, sl] * SCALE + pe_v[s, sl]

    for c in range(NBUF - 1):
        gather_start(c, c)
    pe_copy.wait()

    @pl.loop(0, NCHUNK // NBUF)
    def _(g):
        c0 = g * NBUF
        for j in range(NBUF):
            c = c0 + j
            gather_wait(j)
            compute(c, j)
            out_start(c, j)
            d = c + NBUF - 1
            bd = (j + NBUF - 1) % NBUF
            if j == 0:
                @pl.when(g > 0)
                def _():
                    out_wait(bd)
                    gather_start(d, bd)

                @pl.when(g == 0)
                def _():
                    gather_start(d, bd)
            else:
                @pl.when(d < NCHUNK)
                def _():
                    out_wait(bd)
                    gather_start(d, bd)

    for b in range(NBUF):
        out_wait(b)


@jax.jit
def _embed(table, idx, pe):
    mesh = plsc.VectorSubcoreMesh(
        core_axis_name="c", subcore_axis_name="s", num_cores=NC, num_subcores=NS
    )
    return pl.kernel(
        _sc_body,
        out_type=jax.ShapeDtypeStruct((ROWS, EMB), jnp.float32),
        mesh=mesh,
        scratch_types=[
            pltpu.VMEM((NCHUNK, CHUNK), jnp.int32),
            pltpu.VMEM((S, EMB), jnp.float32),
            pltpu.VMEM((NBUF, CHUNK, EMB), jnp.float32),
        ] + [pltpu.SemaphoreType.DMA] * (2 * NBUF + 1),
    )(table, idx, pe)


def kernel(token_sequences, embedding_weight, positional_embedding):
    idx = token_sequences.astype(jnp.int32).reshape(NW, NCHUNK, CHUNK)
    pe = positional_embedding[0, :S, :]
    out = _embed(embedding_weight, idx, pe)
    return out.reshape(B, S, EMB)
